# baseline (speedup 1.0000x reference)
import functools

import jax
import jax.numpy as jnp
from jax import lax
from jax.experimental import pallas as pl
from jax.experimental.pallas import tpu as pltpu
from jax.experimental.pallas import tpu_sc as plsc

N = 10000
D = 256
C = 128
H = 2
E = 160000

NS = 16
EPT = E // NS
CHUNK = 80
NCH = EPT // CHUNK
SB = 80
KS = CHUNK // SB
RS = 624
TAIL = N - NS * RS
CV = C // 16


def _stage1_body(xu, xi, wu, wi, xl_ui, xr_ui, xl_iu, xr_iu, ru, ri):
    bn = xu.shape[0]
    pu = jnp.dot(xu[...], wu[...], preferred_element_type=jnp.float32)
    pi = jnp.dot(xi[...], wi[...], preferred_element_type=jnp.float32)

    def hm(p):
        return p.reshape(bn, H, C).transpose(1, 0, 2)

    xl_ui[...] = hm(pu[:, :H * C])
    xr_iu[...] = hm(pu[:, H * C:2 * H * C])
    ru[...] = pu[:, 2 * H * C:]
    xl_iu[...] = hm(pi[:, :H * C])
    xr_ui[...] = hm(pi[:, H * C:2 * H * C])
    ri[...] = pi[:, 2 * H * C:]


def _stage1(x_user, x_item, wu, wi):
    bn = 1000
    grid = (N // bn,)
    f32 = jnp.float32
    hm_spec = pl.BlockSpec((H, bn, C), lambda i: (0, i, 0))
    r_spec = pl.BlockSpec((bn, C), lambda i: (i, 0))
    return pl.pallas_call(
        _stage1_body,
        grid=grid,
        in_specs=[
            pl.BlockSpec((bn, D), lambda i: (i, 0)),
            pl.BlockSpec((bn, D), lambda i: (i, 0)),
            pl.BlockSpec((D, 2 * H * C + C), lambda i: (0, 0)),
            pl.BlockSpec((D, 2 * H * C + C), lambda i: (0, 0)),
        ],
        out_specs=[hm_spec, hm_spec, hm_spec, hm_spec, r_spec, r_spec],
        out_shape=[
            jax.ShapeDtypeStruct((H, N, C), f32),
            jax.ShapeDtypeStruct((H, N, C), f32),
            jax.ShapeDtypeStruct((H, N, C), f32),
            jax.ShapeDtypeStruct((H, N, C), f32),
            jax.ShapeDtypeStruct((N, C), f32),
            jax.ShapeDtypeStruct((N, C), f32),
        ],
    )(x_user, x_item, wu, wi)


def _sc_edge_body(xl, xr, src_e, dst_e, attf, num_o, den_o,
                  srcb, dstb, idx_a2, idx_b2, idx_s2,
                  arows2, brows, denr2, attb, sbuf, acc, dacc,
                  sem_a, sem_b, sem_s, sem_i):
    cid = lax.axis_index("c")
    tid = lax.axis_index("s")
    zero16 = jnp.zeros((16,), jnp.float32)

    def z_body(i, carry):
        for c in range(CV):
            arows2[i, pl.ds(c * 16, 16)] = zero16
        denr2[i, pl.ds(0, 16)] = zero16
        return carry

    lax.fori_loop(0, CHUNK, z_body, 0)
    r0 = tid * RS

    def zfill(i, carry):
        pltpu.sync_copy(arows2.at[pl.ds(0, 80)],
                        acc.at[pl.ds(r0 + i * 80, 80)])
        pltpu.sync_copy(denr2.at[pl.ds(0, 80)],
                        dacc.at[pl.ds(r0 + i * 80, 80)])
        return carry

    lax.fori_loop(0, RS // 80, zfill, 0)
    rem = RS - (RS // 80) * 80
    pltpu.sync_copy(arows2.at[pl.ds(0, rem)],
                    acc.at[pl.ds(r0 + RS - rem, rem)])
    pltpu.sync_copy(denr2.at[pl.ds(0, rem)],
                    dacc.at[pl.ds(r0 + RS - rem, rem)])

    @pl.when(tid == NS - 1)
    def _zero_tail():
        pltpu.sync_copy(arows2.at[pl.ds(0, TAIL)],
                        acc.at[pl.ds(NS * RS, TAIL)])
        pltpu.sync_copy(denr2.at[pl.ds(0, TAIL)],
                        dacc.at[pl.ds(NS * RS, TAIL)])

    pltpu.sync_copy(attf.at[pl.ds(cid * C, C)], attb)
    plsc.subcore_barrier()

    att_v = [attb[pl.ds(c * 16, 16)] for c in range(CV)]
    lanes = lax.iota(jnp.int32, 16)
    lane0 = lanes == 0
    cst16 = [jnp.full((16,), j, jnp.int32) for j in range(16)]
    basev = jnp.full((16,), cid * N, jnp.int32)
    ebase = tid * EPT

    def build_idx(q):
        for k in range(CHUNK // 16):
            sv = srcb[pl.ds(k * 16, 16)]
            dv = dstb[pl.ds(k * 16, 16)]
            idx_a2[q, pl.ds(k * 16, 16)] = sv + basev
            idx_b2[q, pl.ds(k * 16, 16)] = dv + basev
            idx_s2[q, pl.ds(k * 16, 16)] = dv

    pltpu.sync_copy(src_e.at[pl.ds(ebase, CHUNK)], srcb)
    pltpu.sync_copy(dst_e.at[pl.ds(ebase, CHUNK)], dstb)
    build_idx(0)
    pltpu.async_copy(xl.at[idx_a2.at[0]], arows2.at[pl.ds(0, CHUNK)], sem_a)

    def chunk_body(i, carry):
        p = jnp.bitwise_and(i, 1)
        q = 1 - p
        pb = p * CHUNK
        qb = q * CHUNK
        cp_b = pltpu.async_copy(xr.at[idx_b2.at[p]], brows, sem_b)

        @pl.when(i < NCH - 1)
        def _prefetch_ids():
            eb2 = ebase + (i + 1) * CHUNK
            pltpu.async_copy(src_e.at[pl.ds(eb2, CHUNK)], srcb, sem_i)
            pltpu.async_copy(dst_e.at[pl.ds(eb2, CHUNK)], dstb, sem_i)

        pltpu.make_async_copy(
            xl.at[idx_a2.at[p]], arows2.at[pl.ds(pb, CHUNK)], sem_a).wait()

        @pl.when(i > 0)
        def _drain_scatter():
            pltpu.make_async_copy(
                xl.at[pl.ds(0, CHUNK)], arows2.at[pl.ds(qb, CHUNK)],
                sem_s).wait()
            pltpu.make_async_copy(
                den_o.at[0].at[pl.ds(0, CHUNK)],
                denr2.at[pl.ds(qb, CHUNK)], sem_s).wait()

        @pl.when(i < NCH - 1)
        def _stage_next():
            pltpu.make_async_copy(
                src_e.at[pl.ds(0, CHUNK)], srcb, sem_i).wait()
            pltpu.make_async_copy(
                dst_e.at[pl.ds(0, CHUNK)], dstb, sem_i).wait()
            build_idx(q)
            pltpu.async_copy(
                xl.at[idx_a2.at[q]], arows2.at[pl.ds(qb, CHUNK)], sem_a)

        cp_b.wait()

        @plsc.parallel_loop(0, CHUNK // 16, 1, unroll=2)
        def group_body(g):
            base = pb + g * 16
            eb0 = g * 16
            for j in range(16):
                row = base + j
                s = zero16
                for c in range(CV):
                    v = (arows2[row, pl.ds(c * 16, 16)]
                         + brows[eb0 + j, pl.ds(c * 16, 16)])
                    s = s + jnp.maximum(v, 0.2 * v) * att_v[c]
                denr2[row, pl.ds(0, 16)] = s
            rows16 = jnp.full((16,), base, jnp.int32) + lanes
            tot = zero16
            for c in range(16):
                tot = tot + plsc.load_gather(denr2, [rows16, cst16[c]])
            exv16 = jnp.exp(tot)
            g_splat = jnp.full((16,), g, jnp.int32)
            sbuf[g, pl.ds(0, 16)] = exv16
            for j in range(16):
                row = base + j
                exj = plsc.load_gather(sbuf, [g_splat, cst16[j]])
                for c in range(CV):
                    arows2[row, pl.ds(c * 16, 16)] = (
                        arows2[row, pl.ds(c * 16, 16)] * exj)
                denr2[row, pl.ds(0, 16)] = jnp.where(lane0, exj, zero16)
        pltpu.async_copy(denr2.at[pl.ds(pb, CHUNK)],
                         dacc.at[idx_s2.at[p]], sem_s, add=True)
        pltpu.async_copy(arows2.at[pl.ds(pb, CHUNK)],
                         acc.at[idx_s2.at[p]], sem_s, add=True)
        return carry

    lax.fori_loop(0, NCH, chunk_body, 0)
    lastb = ((NCH - 1) % 2) * CHUNK
    pltpu.make_async_copy(
        xl.at[pl.ds(0, CHUNK)],
        arows2.at[pl.ds(lastb, CHUNK)], sem_s).wait()
    pltpu.make_async_copy(
        den_o.at[0].at[pl.ds(0, CHUNK)],
        denr2.at[pl.ds(lastb, CHUNK)], sem_s).wait()
    plsc.subcore_barrier()
    pltpu.sync_copy(acc.at[pl.ds(r0, RS)], num_o.at[cid, pl.ds(r0, RS)])
    pltpu.sync_copy(dacc.at[pl.ds(r0, RS)], den_o.at[cid, pl.ds(r0, RS)])

    @pl.when(tid == NS - 1)
    def _copy_tail():
        pltpu.sync_copy(acc.at[pl.ds(NS * RS, TAIL)],
                        num_o.at[cid, pl.ds(NS * RS, TAIL)])
        pltpu.sync_copy(dacc.at[pl.ds(NS * RS, TAIL)],
                        den_o.at[cid, pl.ds(NS * RS, TAIL)])


def _sc_edge(xl_hm, xr_hm, src, dst, attf):
    f32 = jnp.float32
    i32 = jnp.int32
    mesh = plsc.VectorSubcoreMesh(core_axis_name="c", subcore_axis_name="s")
    return pl.kernel(
        _sc_edge_body,
        out_type=[
            jax.ShapeDtypeStruct((H, N, C), f32),
            jax.ShapeDtypeStruct((H, N, 16), f32),
        ],
        mesh=mesh,
        compiler_params=pltpu.CompilerParams(
            needs_layout_passes=False, use_tc_tiling_on_sc=False),
        scratch_types=[
            pltpu.VMEM((CHUNK,), i32),
            pltpu.VMEM((CHUNK,), i32),
            pltpu.VMEM((2, CHUNK), i32),
            pltpu.VMEM((2, CHUNK), i32),
            pltpu.VMEM((2, CHUNK), i32),
            pltpu.VMEM((2 * CHUNK, C), f32),
            pltpu.VMEM((CHUNK, C), f32),
            pltpu.VMEM((2 * CHUNK, 16), f32),
            pltpu.VMEM((C,), f32),
            pltpu.VMEM((CHUNK // 16, 16), f32),
            pltpu.VMEM_SHARED((N, C), f32),
            pltpu.VMEM_SHARED((N, 16), f32),
            pltpu.SemaphoreType.DMA,
            pltpu.SemaphoreType.DMA,
            pltpu.SemaphoreType.DMA,
            pltpu.SemaphoreType.DMA,
        ],
    )(xl_hm, xr_hm, src, dst, attf)


def _stage3_body(num, den, res, b, g, be, out):
    eps = jnp.float32(1e-16)
    d0 = den[0, :, 0:1] + eps
    d1 = den[1, :, 0:1] + eps
    x = 0.5 * (num[0] / d0 + num[1] / d1) + res[...] + b[...]
    mu = jnp.mean(x, axis=0, keepdims=True)
    var = jnp.mean((x - mu) ** 2, axis=0, keepdims=True)
    xn = (x - mu) * lax.rsqrt(var + 1e-5)
    y = g[...] * xn + be[...]
    out[...] = jnp.where(y > 0, y, jnp.exp(y) - 1.0)


def _stage3(num, den, res, b, g, be):
    return pl.pallas_call(
        _stage3_body,
        out_shape=jax.ShapeDtypeStruct((N, C), jnp.float32),
    )(num, den, res, b, g, be)


@jax.jit
def kernel(x_user, x_item, edge_index_ui, edge_index_iu,
           Wl_ui, Wr_ui, att_ui, Wres_ui, b_ui,
           Wl_iu, Wr_iu, att_iu, Wres_iu, b_iu,
           g_user, be_user, g_item, be_item):
    wu = jnp.concatenate([Wl_ui, Wr_iu, Wres_iu], axis=1)
    wi = jnp.concatenate([Wl_iu, Wr_ui, Wres_ui], axis=1)
    xl_ui, xr_ui, xl_iu, xr_iu, res_user, res_item = _stage1(
        x_user, x_item, wu, wi)

    nu_ui, den_ui = _sc_edge(
        xl_ui.reshape(H * N, C), xr_ui.reshape(H * N, C),
        edge_index_ui[0], edge_index_ui[1], att_ui.reshape(H * C))
    nu_iu, den_iu = _sc_edge(
        xl_iu.reshape(H * N, C), xr_iu.reshape(H * N, C),
        edge_index_iu[0], edge_index_iu[1], att_iu.reshape(H * C))

    out_user = _stage3(nu_iu, den_iu, res_user, b_iu.reshape(1, C),
                       g_user.reshape(1, C), be_user.reshape(1, C))
    out_item = _stage3(nu_ui, den_ui, res_item, b_ui.reshape(1, C),
                       g_item.reshape(1, C), be_item.reshape(1, C))
    return (out_user, out_item)


# speedup vs baseline: 2.1194x; 2.1194x over previous
import functools

import jax
import jax.numpy as jnp
from jax import lax
from jax.experimental import pallas as pl
from jax.experimental.pallas import tpu as pltpu
from jax.experimental.pallas import tpu_sc as plsc

N = 10000
D = 256
C = 128
H = 2
E = 160000

NS = 16
EPT = E // NS
CHUNK = 80
NCH = EPT // CHUNK
SB = 80
KS = CHUNK // SB
RS = 624
TAIL = N - NS * RS
CV = C // 16


def _stage1_body(xu, xi, wu, wi, xl_ui, xr_ui, xl_iu, xr_iu, ru, ri):
    bn = xu.shape[0]
    pu = jnp.dot(xu[...], wu[...], preferred_element_type=jnp.float32)
    pi = jnp.dot(xi[...], wi[...], preferred_element_type=jnp.float32)

    def hm(p):
        return p.reshape(bn, H, C).transpose(1, 0, 2)

    xl_ui[...] = hm(pu[:, :H * C])
    xr_iu[...] = hm(pu[:, H * C:2 * H * C])
    ru[...] = pu[:, 2 * H * C:]
    xl_iu[...] = hm(pi[:, :H * C])
    xr_ui[...] = hm(pi[:, H * C:2 * H * C])
    ri[...] = pi[:, 2 * H * C:]


def _stage1(x_user, x_item, wu, wi):
    bn = 1000
    grid = (N // bn,)
    f32 = jnp.float32
    hm_spec = pl.BlockSpec((H, bn, C), lambda i: (0, i, 0))
    r_spec = pl.BlockSpec((bn, C), lambda i: (i, 0))
    return pl.pallas_call(
        _stage1_body,
        grid=grid,
        in_specs=[
            pl.BlockSpec((bn, D), lambda i: (i, 0)),
            pl.BlockSpec((bn, D), lambda i: (i, 0)),
            pl.BlockSpec((D, 2 * H * C + C), lambda i: (0, 0)),
            pl.BlockSpec((D, 2 * H * C + C), lambda i: (0, 0)),
        ],
        out_specs=[hm_spec, hm_spec, hm_spec, hm_spec, r_spec, r_spec],
        out_shape=[
            jax.ShapeDtypeStruct((H, N, C), f32),
            jax.ShapeDtypeStruct((H, N, C), f32),
            jax.ShapeDtypeStruct((H, N, C), f32),
            jax.ShapeDtypeStruct((H, N, C), f32),
            jax.ShapeDtypeStruct((N, C), f32),
            jax.ShapeDtypeStruct((N, C), f32),
        ],
    )(x_user, x_item, wu, wi)


def _sc_edge_body(xl, xr, src_e, dst_e, attf, num_o, den_o,
                  srcb, dstb, idx_a2, idx_b2, idx_s2,
                  arows2, brows, denr2, attb, sbuf, acc, dacc,
                  sem_a, sem_b, sem_s, sem_i):
    cid = lax.axis_index("c")
    tid = lax.axis_index("s")
    zero16 = jnp.zeros((16,), jnp.float32)

    def z_body(i, carry):
        for c in range(CV):
            arows2[i, pl.ds(c * 16, 16)] = zero16
        denr2[i, pl.ds(0, 16)] = zero16
        return carry

    lax.fori_loop(0, CHUNK, z_body, 0)
    r0 = tid * RS

    def zfill(i, carry):
        pltpu.sync_copy(arows2.at[pl.ds(0, 80)],
                        acc.at[pl.ds(r0 + i * 80, 80)])
        pltpu.sync_copy(denr2.at[pl.ds(0, 80)],
                        dacc.at[pl.ds(r0 + i * 80, 80)])
        return carry

    lax.fori_loop(0, RS // 80, zfill, 0)
    rem = RS - (RS // 80) * 80
    pltpu.sync_copy(arows2.at[pl.ds(0, rem)],
                    acc.at[pl.ds(r0 + RS - rem, rem)])
    pltpu.sync_copy(denr2.at[pl.ds(0, rem)],
                    dacc.at[pl.ds(r0 + RS - rem, rem)])

    @pl.when(tid == NS - 1)
    def _zero_tail():
        pltpu.sync_copy(arows2.at[pl.ds(0, TAIL)],
                        acc.at[pl.ds(NS * RS, TAIL)])
        pltpu.sync_copy(denr2.at[pl.ds(0, TAIL)],
                        dacc.at[pl.ds(NS * RS, TAIL)])

    pltpu.sync_copy(attf.at[pl.ds(cid * C, C)], attb)
    plsc.subcore_barrier()

    att_v = [attb[pl.ds(c * 16, 16)] for c in range(CV)]
    lanes = lax.iota(jnp.int32, 16)
    lane0 = lanes == 0
    shuf = [jnp.bitwise_xor(lanes, m) for m in (1, 2, 4, 8)]
    basev = jnp.full((16,), cid * N, jnp.int32)
    ebase = tid * EPT

    def build_idx(q):
        for k in range(CHUNK // 16):
            sv = srcb[pl.ds(k * 16, 16)]
            dv = dstb[pl.ds(k * 16, 16)]
            idx_a2[q, pl.ds(k * 16, 16)] = sv + basev
            idx_b2[q, pl.ds(k * 16, 16)] = dv + basev
            idx_s2[q, pl.ds(k * 16, 16)] = dv

    pltpu.sync_copy(src_e.at[pl.ds(ebase, CHUNK)], srcb)
    pltpu.sync_copy(dst_e.at[pl.ds(ebase, CHUNK)], dstb)
    build_idx(0)
    pltpu.async_copy(xl.at[idx_a2.at[0]], arows2.at[pl.ds(0, CHUNK)], sem_a)

    def chunk_body(i, carry):
        p = jnp.bitwise_and(i, 1)
        q = 1 - p
        pb = p * CHUNK
        qb = q * CHUNK
        cp_b = pltpu.async_copy(xr.at[idx_b2.at[p]], brows, sem_b)

        @pl.when(i < NCH - 1)
        def _prefetch_ids():
            eb2 = ebase + (i + 1) * CHUNK
            pltpu.async_copy(src_e.at[pl.ds(eb2, CHUNK)], srcb, sem_i)
            pltpu.async_copy(dst_e.at[pl.ds(eb2, CHUNK)], dstb, sem_i)

        pltpu.make_async_copy(
            xl.at[idx_a2.at[p]], arows2.at[pl.ds(pb, CHUNK)], sem_a).wait()

        @pl.when(i > 0)
        def _drain_scatter():
            pltpu.make_async_copy(
                xl.at[pl.ds(0, CHUNK)], arows2.at[pl.ds(qb, CHUNK)],
                sem_s).wait()
            pltpu.make_async_copy(
                den_o.at[0].at[pl.ds(0, CHUNK)],
                denr2.at[pl.ds(qb, CHUNK)], sem_s).wait()

        @pl.when(i < NCH - 1)
        def _stage_next():
            pltpu.make_async_copy(
                src_e.at[pl.ds(0, CHUNK)], srcb, sem_i).wait()
            pltpu.make_async_copy(
                dst_e.at[pl.ds(0, CHUNK)], dstb, sem_i).wait()
            build_idx(q)
            pltpu.async_copy(
                xl.at[idx_a2.at[q]], arows2.at[pl.ds(qb, CHUNK)], sem_a)

        cp_b.wait()

        @plsc.parallel_loop(0, CHUNK, 1, unroll=3)
        def edge_body(e):
            row = pb + e
            row_splat = jnp.full((16,), row, jnp.int32)
            avs = []
            s = zero16
            for c in range(CV):
                av = arows2[row, pl.ds(c * 16, 16)]
                avs.append(av)
                v = av + brows[e, pl.ds(c * 16, 16)]
                z = jnp.maximum(v, 0.2 * v)
                s = s + z * att_v[c]
            for sh in shuf:
                denr2[row, pl.ds(0, 16)] = s
                s = s + plsc.load_gather(denr2, [row_splat, sh])
            exv = jnp.exp(s)
            for c in range(CV):
                arows2[row, pl.ds(c * 16, 16)] = avs[c] * exv
            denr2[row, pl.ds(0, 16)] = jnp.where(lane0, exv, zero16)
        pltpu.async_copy(denr2.at[pl.ds(pb, CHUNK)],
                         dacc.at[idx_s2.at[p]], sem_s, add=True)
        pltpu.async_copy(arows2.at[pl.ds(pb, CHUNK)],
                         acc.at[idx_s2.at[p]], sem_s, add=True)
        return carry

    lax.fori_loop(0, NCH, chunk_body, 0)
    lastb = ((NCH - 1) % 2) * CHUNK
    pltpu.make_async_copy(
        xl.at[pl.ds(0, CHUNK)],
        arows2.at[pl.ds(lastb, CHUNK)], sem_s).wait()
    pltpu.make_async_copy(
        den_o.at[0].at[pl.ds(0, CHUNK)],
        denr2.at[pl.ds(lastb, CHUNK)], sem_s).wait()
    plsc.subcore_barrier()
    pltpu.sync_copy(acc.at[pl.ds(r0, RS)], num_o.at[cid, pl.ds(r0, RS)])
    pltpu.sync_copy(dacc.at[pl.ds(r0, RS)], den_o.at[cid, pl.ds(r0, RS)])

    @pl.when(tid == NS - 1)
    def _copy_tail():
        pltpu.sync_copy(acc.at[pl.ds(NS * RS, TAIL)],
                        num_o.at[cid, pl.ds(NS * RS, TAIL)])
        pltpu.sync_copy(dacc.at[pl.ds(NS * RS, TAIL)],
                        den_o.at[cid, pl.ds(NS * RS, TAIL)])


def _sc_edge(xl_hm, xr_hm, src, dst, attf):
    f32 = jnp.float32
    i32 = jnp.int32
    mesh = plsc.VectorSubcoreMesh(core_axis_name="c", subcore_axis_name="s")
    return pl.kernel(
        _sc_edge_body,
        out_type=[
            jax.ShapeDtypeStruct((H, N, C), f32),
            jax.ShapeDtypeStruct((H, N, 16), f32),
        ],
        mesh=mesh,
        compiler_params=pltpu.CompilerParams(
            needs_layout_passes=False, use_tc_tiling_on_sc=False),
        scratch_types=[
            pltpu.VMEM((CHUNK,), i32),
            pltpu.VMEM((CHUNK,), i32),
            pltpu.VMEM((2, CHUNK), i32),
            pltpu.VMEM((2, CHUNK), i32),
            pltpu.VMEM((2, CHUNK), i32),
            pltpu.VMEM((2 * CHUNK, C), f32),
            pltpu.VMEM((CHUNK, C), f32),
            pltpu.VMEM((2 * CHUNK, 16), f32),
            pltpu.VMEM((C,), f32),
            pltpu.VMEM((16,), f32),
            pltpu.VMEM_SHARED((N, C), f32),
            pltpu.VMEM_SHARED((N, 16), f32),
            pltpu.SemaphoreType.DMA,
            pltpu.SemaphoreType.DMA,
            pltpu.SemaphoreType.DMA,
            pltpu.SemaphoreType.DMA,
        ],
    )(xl_hm, xr_hm, src, dst, attf)


def _stage3_body(num, den, res, b, g, be, out):
    eps = jnp.float32(1e-16)
    d0 = den[0, :, 0:1] + eps
    d1 = den[1, :, 0:1] + eps
    x = 0.5 * (num[0] / d0 + num[1] / d1) + res[...] + b[...]
    mu = jnp.mean(x, axis=0, keepdims=True)
    var = jnp.mean((x - mu) ** 2, axis=0, keepdims=True)
    xn = (x - mu) * lax.rsqrt(var + 1e-5)
    y = g[...] * xn + be[...]
    out[...] = jnp.where(y > 0, y, jnp.exp(y) - 1.0)


def _stage3(num, den, res, b, g, be):
    return pl.pallas_call(
        _stage3_body,
        out_shape=jax.ShapeDtypeStruct((N, C), jnp.float32),
    )(num, den, res, b, g, be)


@jax.jit
def kernel(x_user, x_item, edge_index_ui, edge_index_iu,
           Wl_ui, Wr_ui, att_ui, Wres_ui, b_ui,
           Wl_iu, Wr_iu, att_iu, Wres_iu, b_iu,
           g_user, be_user, g_item, be_item):
    wu = jnp.concatenate([Wl_ui, Wr_iu, Wres_iu], axis=1)
    wi = jnp.concatenate([Wl_iu, Wr_ui, Wres_ui], axis=1)
    xl_ui, xr_ui, xl_iu, xr_iu, res_user, res_item = _stage1(
        x_user, x_item, wu, wi)

    nu_ui, den_ui = _sc_edge(
        xl_ui.reshape(H * N, C), xr_ui.reshape(H * N, C),
        edge_index_ui[0], edge_index_ui[1], att_ui.reshape(H * C))
    nu_iu, den_iu = _sc_edge(
        xl_iu.reshape(H * N, C), xr_iu.reshape(H * N, C),
        edge_index_iu[0], edge_index_iu[1], att_iu.reshape(H * C))

    out_user = _stage3(nu_iu, den_iu, res_user, b_iu.reshape(1, C),
                       g_user.reshape(1, C), be_user.reshape(1, C))
    out_item = _stage3(nu_ui, den_ui, res_item, b_ui.reshape(1, C),
                       g_item.reshape(1, C), be_item.reshape(1, C))
    return (out_user, out_item)
